# baseline (speedup 1.0000x reference)
import jax
import jax.numpy as jnp
from jax.experimental import pallas as pl
from jax.experimental.pallas import tpu as pltpu

_TILE = 2048
_SPLIT = 4
_SUB = _TILE // _SPLIT


def _start_in(zf, zbuf, insem, step, slot):
    for k in range(_SPLIT):
        pltpu.make_async_copy(
            zf.at[pl.ds(step * _TILE + k * _SUB, _SUB)],
            zbuf.at[slot, pl.ds(k * _SUB, _SUB)],
            insem.at[slot, k],
        ).start()


def _wait_in(zbuf, insem, slot):
    for k in range(_SPLIT):
        pltpu.make_async_copy(
            zbuf.at[slot, pl.ds(k * _SUB, _SUB)],
            zbuf.at[slot, pl.ds(k * _SUB, _SUB)],
            insem.at[slot, k],
        ).wait()


def _wait_out(obuf, outsem, slot):
    for k in range(_SPLIT):
        pltpu.make_async_copy(
            obuf.at[slot, pl.ds(k * _SUB, _SUB)],
            obuf.at[slot, pl.ds(k * _SUB, _SUB)],
            outsem.at[slot, k],
        ).wait()


def _k_manual(z_hbm, w_ref, b_ref, o_hbm, zbuf, obuf, insem, outsem):
    i = pl.program_id(0)
    steps = pl.num_programs(0)
    slot = jax.lax.rem(i, 2)
    nslot = jax.lax.rem(i + 1, 2)

    @pl.when(i == 0)
    def _():
        _start_in(z_hbm, zbuf, insem, 0, 0)

    @pl.when(i + 1 < steps)
    def _():
        _start_in(z_hbm, zbuf, insem, i + 1, nslot)

    _wait_in(zbuf, insem, slot)

    @pl.when(i >= 2)
    def _():
        _wait_out(obuf, outsem, slot)

    zb = zbuf[slot].reshape(_TILE * 8, zbuf.shape[3])
    acc = jnp.dot(zb, w_ref[...], preferred_element_type=jnp.float32)
    acc = acc + b_ref[...]
    obuf[slot] = acc.astype(obuf.dtype).reshape(_TILE, 8, obuf.shape[3])

    for k in range(_SPLIT):
        pltpu.make_async_copy(
            obuf.at[slot, pl.ds(k * _SUB, _SUB)],
            o_hbm.at[pl.ds(i * _TILE + k * _SUB, _SUB)],
            outsem.at[slot, k],
        ).start()

    @pl.when(i == steps - 1)
    def _():
        _wait_out(obuf, outsem, slot)

    @pl.when(i == steps - 1)
    def _():
        _wait_out(obuf, outsem, nslot)


def kernel(z, w12, b_eff, w_bd, b_bd):
    B, in_dim = z.shape
    out_dim = w12.shape[1]
    b = b_eff.reshape(1, out_dim)

    rows = B // 8
    zv = z.reshape(rows, 8, in_dim)
    steps = rows // _TILE
    out = pl.pallas_call(
        _k_manual,
        out_shape=jax.ShapeDtypeStruct((rows, 8, out_dim), z.dtype),
        grid=(steps,),
        in_specs=[
            pl.BlockSpec(memory_space=pl.ANY),
            pl.BlockSpec((in_dim, out_dim), lambda i: (0, 0)),
            pl.BlockSpec((1, out_dim), lambda i: (0, 0)),
        ],
        out_specs=pl.BlockSpec(memory_space=pl.ANY),
        scratch_shapes=[
            pltpu.VMEM((2, _TILE, 8, in_dim), jnp.float32),
            pltpu.VMEM((2, _TILE, 8, out_dim), jnp.float32),
            pltpu.SemaphoreType.DMA((2, _SPLIT)),
            pltpu.SemaphoreType.DMA((2, _SPLIT)),
        ],
        compiler_params=pltpu.CompilerParams(
            dimension_semantics=("arbitrary",),
            vmem_limit_bytes=60 * 1024 * 1024,
        ),
    )(zv, w12, b)

    return out.reshape(B, out_dim)


# speedup vs baseline: 1.0028x; 1.0028x over previous
import jax
import jax.numpy as jnp
from jax.experimental import pallas as pl
from jax.experimental.pallas import tpu as pltpu


def _k3d(z_ref, w_ref, b_ref, o_ref):
    t = z_ref.shape[0]
    zb = z_ref[...].reshape(t * 8, z_ref.shape[2])
    acc = jnp.dot(zb, w_ref[...], preferred_element_type=jnp.float32)
    acc = acc + b_ref[...]
    o_ref[...] = acc.astype(o_ref.dtype).reshape(t, 8, o_ref.shape[2])


def _k2d(z_ref, w_ref, b_ref, o_ref):
    acc = jnp.dot(z_ref[...], w_ref[...], preferred_element_type=jnp.float32)
    o_ref[...] = (acc + b_ref[...]).astype(o_ref.dtype)


def kernel(z, w12, b_eff, w_bd, b_bd):
    B, in_dim = z.shape
    out_dim = w12.shape[1]
    b = b_eff.reshape(1, out_dim)

    if B % 8 != 0:
        tile = min(B, 8192)
        return pl.pallas_call(
            _k2d,
            out_shape=jax.ShapeDtypeStruct((B, out_dim), z.dtype),
            grid=(pl.cdiv(B, tile),),
            in_specs=[
                pl.BlockSpec((tile, in_dim), lambda i: (i, 0)),
                pl.BlockSpec((in_dim, out_dim), lambda i: (0, 0)),
                pl.BlockSpec((1, out_dim), lambda i: (0, 0)),
            ],
            out_specs=pl.BlockSpec((tile, out_dim), lambda i: (i, 0)),
            compiler_params=pltpu.CompilerParams(
                dimension_semantics=("parallel",),
                vmem_limit_bytes=60 * 1024 * 1024,
            ),
        )(z, w12, b)

    rows = B // 8
    zv = z.reshape(rows, 8, in_dim)
    tile = 2048 if rows % 2048 == 0 else 8 * max(1, rows // 64)
    steps = pl.cdiv(rows, tile)
    out = pl.pallas_call(
        _k3d,
        out_shape=jax.ShapeDtypeStruct((rows, 8, out_dim), z.dtype),
        grid=(steps,),
        in_specs=[
            pl.BlockSpec((tile, 8, in_dim), lambda i: (i, 0, 0)),
            pl.BlockSpec((in_dim, out_dim), lambda i: (0, 0)),
            pl.BlockSpec((1, out_dim), lambda i: (0, 0)),
        ],
        out_specs=pl.BlockSpec((tile, 8, out_dim), lambda i: (i, 0, 0)),
        compiler_params=pltpu.CompilerParams(
            dimension_semantics=("parallel",),
            vmem_limit_bytes=60 * 1024 * 1024,
        ),
    )(zv, w12, b)

    return out.reshape(B, out_dim)
